# baseline (speedup 1.0000x reference)
import functools

import jax
import jax.numpy as jnp
from jax import lax
from jax.experimental import pallas as pl
from jax.experimental.pallas import tpu as pltpu
from jax.experimental.pallas import tpu_sc as plsc

_N_SLOTS = 100000
_D_STATE = 8
_NGRAM = 16
_N_BLADES = 8

_NC = 2
_NS = 16
_LANES = 16

_B = 16
_S = 4096
_BW_CHUNK = 256
_N_CHUNKS = _S // _BW_CHUNK
_HASH_UNROLL = 8
_GATHER_UNROLL = 32


def _sc_hash_gather(bw_t, bank_t):
    mesh = plsc.VectorSubcoreMesh(
        core_axis_name="c", subcore_axis_name="s",
        num_cores=_NC, num_subcores=_NS)

    @functools.partial(
        pl.kernel,
        compiler_params=pltpu.CompilerParams(
            needs_layout_passes=False, use_tc_tiling_on_sc=True,
            disable_bounds_checks=True),
        out_type=jax.ShapeDtypeStruct((_B, _N_BLADES, _D_STATE, _S),
                                      jnp.float32),
        mesh=mesh,
        scratch_types=[
            pltpu.VMEM((_N_SLOTS,), jnp.float32),
            pltpu.VMEM((_NGRAM, _BW_CHUNK), jnp.int32),
            pltpu.VMEM((_NGRAM, _BW_CHUNK), jnp.int32),
            pltpu.VMEM((_S,), jnp.int32),
            pltpu.VMEM((_S,), jnp.int32),
            pltpu.VMEM((_S,), jnp.float32),
            pltpu.VMEM((_S,), jnp.float32),
            pltpu.VMEM_SHARED((_B * _S,), jnp.int32),
            pltpu.SemaphoreType.DMA,
            pltpu.SemaphoreType.DMA,
            pltpu.SemaphoreType.DMA,
            pltpu.SemaphoreType.DMA,
            pltpu.SemaphoreType.DMA,
        ],
    )
    def k(bw_hbm, bank_hbm, out_hbm, table_v, bw_v0, bw_v1,
          addr_v0, addr_v1, ov0, ov1, addr_sh,
          sem_bw, sem_addr, sem_o0, sem_o1, sem_tbl):
        bw_b = (bw_v0, bw_v1)
        addr_b = (addr_v0, addr_v1)
        ov_b = (ov0, ov1)
        cid = lax.axis_index("c")
        sid = lax.axis_index("s")
        w = cid * _NS + sid

        def tbl_cp(t):
            p = w * 2 + t
            return pltpu.make_async_copy(
                bank_hbm.at[p // _D_STATE, p % _D_STATE, :], table_v,
                sem_tbl)

        tbl_cp(0).start()

        def bw_cp(c, buf):
            return pltpu.make_async_copy(
                bw_hbm.at[sid, :, pl.ds(c * _BW_CHUNK, _BW_CHUNK)],
                bw_b[buf], sem_bw)

        def hash_chunk(c, buf):
            @plsc.parallel_loop(0, _BW_CHUNK // _LANES,
                                unroll=_HASH_UNROLL)
            def _(g):
                t0 = g * _LANES
                h = jnp.full((_LANES,), 2166136261, dtype=jnp.uint32)
                for i in range(_NGRAM):
                    byte = bw_b[buf][i, pl.ds(t0, _LANES)]
                    h = ((h ^ byte.astype(jnp.uint32))
                         * jnp.uint32(16777619))
                addr = (h % jnp.uint32(_N_SLOTS)).astype(jnp.int32)
                addr_v0[pl.ds(c * _BW_CHUNK + t0, _LANES)] = addr

        bw_cp(0, 0).start()

        def chunk_pair(j, carry):
            c0 = 2 * j
            bw_cp(c0, 0).wait()
            bw_cp(c0 + 1, 1).start()
            hash_chunk(c0, 0)
            bw_cp(c0 + 1, 1).wait()

            @pl.when(j + 1 < _N_CHUNKS // 2)
            def _():
                bw_cp(c0 + 2, 0).start()
            hash_chunk(c0 + 1, 1)
            return carry
        lax.fori_loop(0, _N_CHUNKS // 2, chunk_pair, 0)
        pltpu.sync_copy(addr_v0, addr_sh.at[pl.ds(sid * _S, _S)])
        plsc.subcore_barrier()

        for t in range(2):
            p = w * 2 + t
            blade = p // _D_STATE
            d = p % _D_STATE

            def addr_cp(b, buf):
                return pltpu.make_async_copy(
                    addr_sh.at[pl.ds(b * _S, _S)], addr_b[buf],
                    sem_addr)

            def out_cp(b, buf, blade=blade, d=d):
                return pltpu.make_async_copy(
                    ov_b[buf], out_hbm.at[b, blade, d, :],
                    sem_o0 if buf == 0 else sem_o1)

            def gather_row(buf):
                @plsc.parallel_loop(0, _S // _LANES,
                                    unroll=_GATHER_UNROLL)
                def _(g):
                    t0 = g * _LANES
                    idx = addr_b[buf][pl.ds(t0, _LANES)]
                    ov_b[buf][pl.ds(t0, _LANES)] = (
                        plsc.load_gather(table_v, [idx]))

            if t == 1:
                tbl_cp(1).start()
            tbl_cp(t).wait()
            addr_cp(0, 0).start()

            def row_pair(j, carry, t=t):
                b0 = 2 * j
                addr_cp(b0, 0).wait()
                addr_cp(b0 + 1, 1).start()

                @pl.when(j >= 1)
                def _():
                    out_cp(b0 - 2, 0).wait()
                gather_row(0)
                out_cp(b0, 0).start()
                addr_cp(b0 + 1, 1).wait()

                @pl.when(j + 1 < _B // 2)
                def _():
                    addr_cp(b0 + 2, 0).start()

                @pl.when(j >= 1)
                def _():
                    out_cp(b0 - 1, 1).wait()
                gather_row(1)
                out_cp(b0 + 1, 1).start()
                return carry
            lax.fori_loop(0, _B // 2, row_pair, 0)
            out_cp(_B - 2, 0).wait()
            out_cp(_B - 1, 1).wait()

    return k(bw_t, bank_t)


def kernel(byte_window, bank):
    bw_t = jnp.transpose(byte_window, (0, 2, 1))
    bank_t = jnp.transpose(bank, (0, 2, 1))
    out_t = _sc_hash_gather(bw_t, bank_t)
    return jnp.transpose(out_t, (0, 3, 1, 2))


# speedup vs baseline: 1.0159x; 1.0159x over previous
import functools

import jax
import jax.numpy as jnp
from jax import lax
from jax.experimental import pallas as pl
from jax.experimental.pallas import tpu as pltpu
from jax.experimental.pallas import tpu_sc as plsc

_N_SLOTS = 100000
_D_STATE = 8
_NGRAM = 16
_N_BLADES = 8

_NC = 2
_NS = 16
_LANES = 16

_B = 16
_S = 4096
_BW_CHUNK = 256
_N_CHUNKS = _S // _BW_CHUNK
_HASH_UNROLL = 8
_GATHER_UNROLL = 16


def _sc_hash_gather(bw_t, bank_t):
    mesh = plsc.VectorSubcoreMesh(
        core_axis_name="c", subcore_axis_name="s",
        num_cores=_NC, num_subcores=_NS)

    @functools.partial(
        pl.kernel,
        compiler_params=pltpu.CompilerParams(
            needs_layout_passes=False, use_tc_tiling_on_sc=True,
            disable_bounds_checks=True),
        out_type=jax.ShapeDtypeStruct((_B, _N_BLADES, _D_STATE, _S),
                                      jnp.float32),
        mesh=mesh,
        scratch_types=[
            pltpu.VMEM((_N_SLOTS,), jnp.float32),
            pltpu.VMEM((_NGRAM, _BW_CHUNK), jnp.int32),
            pltpu.VMEM((_NGRAM, _BW_CHUNK), jnp.int32),
            pltpu.VMEM((_S,), jnp.int32),
            pltpu.VMEM((_S,), jnp.int32),
            pltpu.VMEM((_S,), jnp.float32),
            pltpu.VMEM((_S,), jnp.float32),
            pltpu.VMEM_SHARED((_B * _S,), jnp.int32),
            pltpu.SemaphoreType.DMA,
            pltpu.SemaphoreType.DMA,
            pltpu.SemaphoreType.DMA,
            pltpu.SemaphoreType.DMA,
            pltpu.SemaphoreType.DMA,
        ],
    )
    def k(bw_hbm, bank_hbm, out_hbm, table_v, bw_v0, bw_v1,
          addr_v0, addr_v1, ov0, ov1, addr_sh,
          sem_bw, sem_addr, sem_o0, sem_o1, sem_tbl):
        bw_b = (bw_v0, bw_v1)
        addr_b = (addr_v0, addr_v1)
        ov_b = (ov0, ov1)
        cid = lax.axis_index("c")
        sid = lax.axis_index("s")
        w = cid * _NS + sid

        def tbl_cp(t):
            p = w * 2 + t
            return pltpu.make_async_copy(
                bank_hbm.at[p // _D_STATE, p % _D_STATE, :], table_v,
                sem_tbl)

        tbl_cp(0).start()

        def bw_cp(c, buf):
            return pltpu.make_async_copy(
                bw_hbm.at[sid, :, pl.ds(c * _BW_CHUNK, _BW_CHUNK)],
                bw_b[buf], sem_bw)

        def hash_chunk(c, buf):
            @plsc.parallel_loop(0, _BW_CHUNK // _LANES,
                                unroll=_HASH_UNROLL)
            def _(g):
                t0 = g * _LANES
                h = jnp.full((_LANES,), 2166136261, dtype=jnp.uint32)
                for i in range(_NGRAM):
                    byte = bw_b[buf][i, pl.ds(t0, _LANES)]
                    h = ((h ^ byte.astype(jnp.uint32))
                         * jnp.uint32(16777619))
                addr = (h % jnp.uint32(_N_SLOTS)).astype(jnp.int32)
                addr_v0[pl.ds(c * _BW_CHUNK + t0, _LANES)] = addr

        bw_cp(0, 0).start()

        def chunk_pair(j, carry):
            c0 = 2 * j
            bw_cp(c0, 0).wait()
            bw_cp(c0 + 1, 1).start()
            hash_chunk(c0, 0)
            bw_cp(c0 + 1, 1).wait()

            @pl.when(j + 1 < _N_CHUNKS // 2)
            def _():
                bw_cp(c0 + 2, 0).start()
            hash_chunk(c0 + 1, 1)
            return carry
        lax.fori_loop(0, _N_CHUNKS // 2, chunk_pair, 0)
        pltpu.sync_copy(addr_v0, addr_sh.at[pl.ds(sid * _S, _S)])
        plsc.subcore_barrier()

        for t in range(2):
            p = w * 2 + t
            blade = p // _D_STATE
            d = p % _D_STATE

            def addr_cp(b, buf):
                return pltpu.make_async_copy(
                    addr_sh.at[pl.ds(b * _S, _S)], addr_b[buf],
                    sem_addr)

            def out_cp(b, buf, blade=blade, d=d):
                return pltpu.make_async_copy(
                    ov_b[buf], out_hbm.at[b, blade, d, :],
                    sem_o0 if buf == 0 else sem_o1)

            def gather_row(buf):
                @plsc.parallel_loop(0, _S // _LANES,
                                    unroll=_GATHER_UNROLL)
                def _(g):
                    t0 = g * _LANES
                    idx = addr_b[buf][pl.ds(t0, _LANES)]
                    ov_b[buf][pl.ds(t0, _LANES)] = (
                        plsc.load_gather(table_v, [idx]))

            if t == 1:
                tbl_cp(1).start()
            addr_cp(0, 0).start()
            tbl_cp(t).wait()

            def row_pair(j, carry, t=t):
                b0 = 2 * j
                addr_cp(b0, 0).wait()
                addr_cp(b0 + 1, 1).start()

                @pl.when(j >= 1)
                def _():
                    out_cp(b0 - 2, 0).wait()
                gather_row(0)
                out_cp(b0, 0).start()
                addr_cp(b0 + 1, 1).wait()

                @pl.when(j + 1 < _B // 2)
                def _():
                    addr_cp(b0 + 2, 0).start()

                @pl.when(j >= 1)
                def _():
                    out_cp(b0 - 1, 1).wait()
                gather_row(1)
                out_cp(b0 + 1, 1).start()
                return carry
            lax.fori_loop(0, _B // 2, row_pair, 0)
            out_cp(_B - 2, 0).wait()
            out_cp(_B - 1, 1).wait()

    return k(bw_t, bank_t)


def kernel(byte_window, bank):
    bw_t = jnp.transpose(byte_window, (0, 2, 1))
    bank_t = jnp.transpose(bank, (0, 2, 1))
    out_t = _sc_hash_gather(bw_t, bank_t)
    return jnp.transpose(out_t, (0, 3, 1, 2))
